# baseline (speedup 1.0000x reference)
import functools

import jax
import jax.numpy as jnp
import numpy as np
from jax import lax
from jax.experimental import pallas as pl
from jax.experimental.pallas import tpu as pltpu
from jax.experimental.pallas import tpu_sc as plsc

S = 8192
NLEV = 13
W = 32
WG = 128
KIN = 6
KWMAX = 64
DEPTH = 2
EB = 256

NC = 2
NS = 16
NW = NC * NS


def _nl(l):
    return S >> l


def _el(l):
    return max((S >> l) * 16, 32)


def _kwl(l):
    return max(KWMAX >> l, 16)


_PHI_SLOT = [max(_nl(l), 8) for l in range(NLEV)]
_PHI_OFF = np.concatenate([[0], np.cumsum(_PHI_SLOT)]).astype(np.int64)
P_TOT = int(_PHI_OFF[-1])

_APPS = [(NLEV - 1, NLEV - 1, _nl(NLEV - 1))]
_APPS += [(l, l - 1, _nl(l - 1)) for l in range(NLEV - 1, 0, -1)]
_APPS += [(0, 0, _nl(0))]

_EPAD = [((_el(l) + EB - 1) // EB) * EB for l in range(NLEV)]
_APP_ESLOT = [_EPAD[a[0]] for a in _APPS]
_APP_EOFF = np.concatenate([[0], np.cumsum(_APP_ESLOT)]).astype(np.int64)
E_TOT = int(_APP_EOFF[-1])
NB = E_TOT // EB

_AGG_SLOT = [max(a[2], 8) for a in _APPS]
_AGG_OFF = np.concatenate([[0], np.cumsum(_AGG_SLOT)]).astype(np.int64)
TRASH = int(_AGG_OFF[-1])
A_TOT = ((TRASH + 8 + 127) // 128) * 128

PW = E_TOT // NW
CH = 128
NCH = PW // CH
TAIL = PW - NCH * CH
NCHK = E_TOT // CH
ROUNDS = (NCHK + NW - 1) // NW
SLICE = A_TOT // NS
NTC = SLICE // CH
TR = SLICE - NTC * CH

_LMAP = np.concatenate(
    [np.full(_APP_ESLOT[a] // EB, _APPS[a][0], np.int32) for a in range(len(_APPS))]
)

_F32 = jnp.float32


def _fc1_body(x_ref, w_ref, b_ref, o_ref):
    o_ref[...] = (
        jnp.dot(x_ref[...], w_ref[...], preferred_element_type=_F32) + b_ref[...]
    )


def _fc1_call(x, w, b):
    return pl.pallas_call(
        _fc1_body,
        out_shape=jax.ShapeDtypeStruct((S, W), _F32),
    )(x, w, b.reshape(1, W))


def _phi_body(x_ref, phi_ref):
    cur = x_ref[...]
    for l in range(NLEV):
        n = _nl(l)
        o = int(_PHI_OFF[l])
        phi_ref[o : o + n, 0:W] = cur
        if l < NLEV - 1:
            cur = jnp.mean(cur.reshape(n // 2, 2, W), axis=1)
        else:
            phi_ref[o + n : o + 8, 0:W] = jnp.zeros((8 - n, W), _F32)


def _phi_call(x):
    return pl.pallas_call(
        _phi_body,
        out_shape=jax.ShapeDtypeStruct((P_TOT, WG), _F32),
    )(x)


def _edge_body(lm_ref, ea_ref, xs_ref, w1_ref, b1_ref, w2_ref, b2_ref,
               w3_ref, b3_ref, msg_ref):
    h = jnp.dot(ea_ref[...], w1_ref[0], preferred_element_type=_F32) + b1_ref[0]
    h = jnp.maximum(h, 0.0)
    h = jnp.dot(h, w2_ref[0], preferred_element_type=_F32) + b2_ref[0]
    h = jnp.maximum(h, 0.0)
    wm = jnp.dot(h, w3_ref[0], preferred_element_type=_F32) + b3_ref[0]
    xs = xs_ref[:, 0:W]
    acc = xs[:, 0:1] * wm[:, 0:W]
    for i in range(1, W):
        acc = acc + xs[:, i : i + 1] * wm[:, i * W : (i + 1) * W]
    msg_ref[...] = acc


def _edge_call(lmap, ea, xs, w1s, b1s, w2s, b2s, w3s, b3s):
    grid_spec = pltpu.PrefetchScalarGridSpec(
        num_scalar_prefetch=1,
        grid=(NB,),
        in_specs=[
            pl.BlockSpec((EB, KIN), lambda b, lm: (b, 0)),
            pl.BlockSpec((EB, WG), lambda b, lm: (b, 0)),
            pl.BlockSpec((1, KIN, KWMAX), lambda b, lm: (lm[b], 0, 0)),
            pl.BlockSpec((1, 1, KWMAX), lambda b, lm: (lm[b], 0, 0)),
            pl.BlockSpec((1, KWMAX, KWMAX), lambda b, lm: (lm[b], 0, 0)),
            pl.BlockSpec((1, 1, KWMAX), lambda b, lm: (lm[b], 0, 0)),
            pl.BlockSpec((1, KWMAX, W * W), lambda b, lm: (lm[b], 0, 0)),
            pl.BlockSpec((1, 1, W * W), lambda b, lm: (lm[b], 0, 0)),
        ],
        out_specs=pl.BlockSpec((EB, W), lambda b, lm: (b, 0)),
    )
    return pl.pallas_call(
        _edge_body,
        grid_spec=grid_spec,
        out_shape=jax.ShapeDtypeStruct((E_TOT, W), _F32),
        compiler_params=pltpu.CompilerParams(
            dimension_semantics=("arbitrary",),
        ),
    )(lmap, ea, xs, w1s, b1s, w2s, b2s, w3s, b3s)


_A_SMALL = int(_AGG_OFF[11])
_P_SMALL = int(_PHI_OFF[2])


def _combine_small_body(agg_ref, cnt_ref, phi_ref, root_ref, cb_ref, out_ref):
    po = int(_PHI_OFF[NLEV - 1]) - _P_SMALL
    x = phi_ref[po : po + _nl(NLEV - 1), 0:W]
    for a in range(11):
        lvl, srcl, n_out = _APPS[a]
        ao = int(_AGG_OFF[a])
        agg = agg_ref[0, ao : ao + n_out, :] + agg_ref[1, ao : ao + n_out, :]
        cnt = cnt_ref[0, ao : ao + n_out, 0:1] + cnt_ref[1, ao : ao + n_out, 0:1]
        agg = agg / jnp.maximum(cnt, 1.0)
        so = int(_PHI_OFF[srcl]) - _P_SMALL
        xin = phi_ref[so : so + n_out, 0:W]
        delta = (
            agg
            + jnp.dot(xin, root_ref[lvl], preferred_element_type=_F32)
            + cb_ref[lvl]
        )
        if a > 0:
            x = jnp.broadcast_to(x[:, None, :], (n_out // 2, 2, W)).reshape(n_out, W)
        x = jnp.maximum(x + delta, 0.0)
    out_ref[...] = x


def _combine_one_body(x_ref, agg_ref, cnt_ref, phi_ref, root_ref, cb_ref,
                      out_ref, *, n_out, up):
    agg = agg_ref[0] + agg_ref[1]
    cnt = cnt_ref[0, :, 0:1] + cnt_ref[1, :, 0:1]
    agg = agg / jnp.maximum(cnt, 1.0)
    delta = (
        agg
        + jnp.dot(phi_ref[:, 0:W], root_ref[...], preferred_element_type=_F32)
        + cb_ref[...]
    )
    x = x_ref[...]
    if up:
        x = jnp.broadcast_to(x[:, None, :], (n_out // 2, 2, W)).reshape(n_out, W)
    out_ref[...] = jnp.maximum(x + delta, 0.0)


def _combine_call(agg, cnt, phi, roots, cbias):
    x = pl.pallas_call(
        _combine_small_body,
        out_shape=jax.ShapeDtypeStruct((2048, W), _F32),
    )(agg[:, :_A_SMALL], cnt[:, :_A_SMALL], phi[_P_SMALL:], roots, cbias)
    for a in (11, 12, 13):
        lvl, srcl, n_out = _APPS[a]
        ao = int(_AGG_OFF[a])
        so = int(_PHI_OFF[srcl])
        body = functools.partial(_combine_one_body, n_out=n_out, up=(lvl != 0))
        x = pl.pallas_call(
            body,
            out_shape=jax.ShapeDtypeStruct((n_out, W), _F32),
        )(x, agg[:, ao : ao + n_out], cnt[:, ao : ao + n_out],
          phi[so : so + n_out], roots[lvl], cbias[lvl])
    return x


def _fc23_body(x_ref, w2_ref, b2_ref, w3_ref, b3_ref, o_ref):
    h = jnp.dot(x_ref[...], w2_ref[...], preferred_element_type=_F32) + b2_ref[...]
    h = jnp.maximum(h, 0.0)
    o_ref[...] = jnp.dot(h, w3_ref[...], preferred_element_type=_F32) + b3_ref[...]


def _fc23_call(x, w2, b2, w3, b3):
    return pl.pallas_call(
        _fc23_body,
        out_shape=jax.ShapeDtypeStruct((S, 1), _F32),
    )(x, w2, b2.reshape(1, -1), w3, b3.reshape(1, 1))


def _sc_mesh():
    return plsc.VectorSubcoreMesh(core_axis_name="c", subcore_axis_name="s")


def _sc_gather(phi, gsrc):
    @functools.partial(
        pl.kernel,
        mesh=_sc_mesh(),
        out_type=jax.ShapeDtypeStruct((E_TOT, WG), _F32),
        scratch_types=[
            pltpu.VMEM((CH,), jnp.int32),
            pltpu.VMEM((CH, WG), _F32),
            pltpu.VMEM((TAIL,), jnp.int32),
            pltpu.VMEM((TAIL, WG), _F32),
            pltpu.SemaphoreType.DMA,
        ],
    )
    def k(phi_hbm, idx_hbm, out_hbm, idx_v, rows_v, idxt_v, rowst_v, sem):
        wid = lax.axis_index("s") * NC + lax.axis_index("c")
        base = wid * PW

        def chunk(i, carry):
            off = base + i * CH
            pltpu.sync_copy(idx_hbm.at[pl.ds(off, CH)], idx_v)
            pltpu.async_copy(phi_hbm.at[idx_v], rows_v, sem).wait()
            pltpu.sync_copy(rows_v, out_hbm.at[pl.ds(off, CH)])
            return carry

        lax.fori_loop(0, NCH, chunk, 0)
        offt = base + NCH * CH
        pltpu.sync_copy(idx_hbm.at[pl.ds(offt, TAIL)], idxt_v)
        pltpu.async_copy(phi_hbm.at[idxt_v], rowst_v, sem).wait()
        pltpu.sync_copy(rowst_v, out_hbm.at[pl.ds(offt, TAIL)])

    return k(phi, gsrc)


def _sc_scatter(msg, gdst, zacc):
    @functools.partial(
        pl.kernel,
        mesh=_sc_mesh(),
        out_type=jax.ShapeDtypeStruct((NC * A_TOT, W), _F32),
        scratch_types=[
            pltpu.VMEM((CH,), jnp.int32),
            pltpu.VMEM((CH, W), _F32),
            pltpu.VMEM((CH, W), _F32),
            pltpu.VMEM_SHARED((A_TOT, W), _F32),
        ],
    )
    def k(msg_hbm, dst_hbm, z_hbm, out_hbm, idx_v, rows_v, stg_v, acc_sh):
        c = lax.axis_index("c")
        s = lax.axis_index("s")
        wid = s * NC + c
        ro = s * SLICE
        pltpu.sync_copy(z_hbm, stg_v)
        for j in range(NTC):
            pltpu.sync_copy(stg_v, acc_sh.at[pl.ds(ro + j * CH, CH)])
        pltpu.sync_copy(
            stg_v.at[pl.ds(0, TR)], acc_sh.at[pl.ds(ro + NTC * CH, TR)]
        )
        plsc.subcore_barrier()

        def chunk(i, carry):
            cid = i * NW + wid

            @pl.when(cid < NCHK)
            def _():
                off = cid * CH
                pltpu.sync_copy(dst_hbm.at[pl.ds(off, CH)], idx_v)
                pltpu.sync_copy(msg_hbm.at[pl.ds(off, CH)], rows_v)
                pltpu.sync_copy(rows_v, acc_sh.at[idx_v], add=True)

            return carry

        lax.fori_loop(0, ROUNDS, chunk, 0)

        plsc.subcore_barrier()
        for j in range(NTC):
            pltpu.sync_copy(acc_sh.at[pl.ds(ro + j * CH, CH)], stg_v)
            pltpu.sync_copy(
                stg_v, out_hbm.at[pl.ds(c * A_TOT + ro + j * CH, CH)]
            )
        pltpu.sync_copy(
            acc_sh.at[pl.ds(ro + NTC * CH, TR)], stg_v.at[pl.ds(0, TR)]
        )
        pltpu.sync_copy(
            stg_v.at[pl.ds(0, TR)],
            out_hbm.at[pl.ds(c * A_TOT + ro + NTC * CH, TR)],
        )

    return k(msg, gdst, zacc)


def _sc_count(gdst, onerow, zacc16):
    @functools.partial(
        pl.kernel,
        mesh=_sc_mesh(),
        out_type=jax.ShapeDtypeStruct((NC * A_TOT, 16), _F32),
        scratch_types=[
            pltpu.VMEM((CH,), jnp.int32),
            pltpu.VMEM((CH, 16), _F32),
            pltpu.VMEM((CH, 16), _F32),
            pltpu.VMEM_SHARED((A_TOT, 16), _F32),
        ],
    )
    def k(dst_hbm, one_hbm, z_hbm, out_hbm, idx_v, one_v, stg_v, acc_sh):
        c = lax.axis_index("c")
        s = lax.axis_index("s")
        wid = s * NC + c
        ro = s * SLICE
        pltpu.sync_copy(one_hbm, one_v)
        pltpu.sync_copy(z_hbm, stg_v)
        for j in range(NTC):
            pltpu.sync_copy(stg_v, acc_sh.at[pl.ds(ro + j * CH, CH)])
        pltpu.sync_copy(
            stg_v.at[pl.ds(0, TR)], acc_sh.at[pl.ds(ro + NTC * CH, TR)]
        )
        plsc.subcore_barrier()

        def chunk(i, carry):
            cid = i * NW + wid

            @pl.when(cid < NCHK)
            def _():
                pltpu.sync_copy(dst_hbm.at[pl.ds(cid * CH, CH)], idx_v)
                pltpu.sync_copy(one_v, acc_sh.at[idx_v], add=True)

            return carry

        lax.fori_loop(0, ROUNDS, chunk, 0)

        plsc.subcore_barrier()
        for j in range(NTC):
            pltpu.sync_copy(acc_sh.at[pl.ds(ro + j * CH, CH)], stg_v)
            pltpu.sync_copy(
                stg_v, out_hbm.at[pl.ds(c * A_TOT + ro + j * CH, CH)]
            )
        pltpu.sync_copy(
            acc_sh.at[pl.ds(ro + NTC * CH, TR)], stg_v.at[pl.ds(0, TR)]
        )
        pltpu.sync_copy(
            stg_v.at[pl.ds(0, TR)],
            out_hbm.at[pl.ds(c * A_TOT + ro + NTC * CH, TR)],
        )

    return k(gdst, onerow, zacc16)


def _pad_to(x, rows, val=0):
    pad = rows - x.shape[0]
    if pad == 0:
        return x
    cfg = [(0, pad)] + [(0, 0)] * (x.ndim - 1)
    return jnp.pad(x, cfg, constant_values=val)


def kernel(X_list, edge_index_list, edge_attr_list, params):
    gsrc, gdst, eas = [], [], []
    for a, (lvl, srcl, n_out) in enumerate(_APPS):
        slot = _APP_ESLOT[a]
        src = jnp.asarray(edge_index_list[lvl][0], jnp.int32)
        dst = jnp.asarray(edge_index_list[lvl][1], jnp.int32)
        gsrc.append(_pad_to(src + int(_PHI_OFF[srcl]), slot, 0))
        gdst.append(_pad_to(dst + int(_AGG_OFF[a]), slot, TRASH))
        eas.append(_pad_to(edge_attr_list[lvl].astype(_F32), slot))
    gsrc = jnp.concatenate(gsrc)
    gdst = jnp.concatenate(gdst)
    ea = jnp.concatenate(eas)

    convs = params["convs"]
    w1s = jnp.stack([
        _pad_to(c["mlp"][0]["w"].T, KWMAX).T for c in convs
    ])
    b1s = jnp.stack([_pad_to(c["mlp"][0]["b"], KWMAX) for c in convs])[:, None, :]
    w2s = jnp.stack([
        _pad_to(_pad_to(c["mlp"][1]["w"].T, KWMAX).T, KWMAX) for c in convs
    ])
    b2s = jnp.stack([_pad_to(c["mlp"][1]["b"], KWMAX) for c in convs])[:, None, :]
    w3s = jnp.stack([_pad_to(c["mlp"][2]["w"], KWMAX) for c in convs])
    b3s = jnp.stack([c["mlp"][2]["b"] for c in convs])[:, None, :]
    roots = jnp.stack([c["root"] for c in convs])
    cbias = jnp.stack([c["bias"] for c in convs])[:, None, :]

    lmap = jnp.asarray(_LMAP)
    zacc = jnp.zeros((CH, W), _F32)
    zacc16 = jnp.zeros((CH, 16), _F32)
    onerow = jnp.ones((CH, 16), _F32)

    c1 = jnp.zeros((A_TOT, 16), _F32).at[gdst].add(
        jnp.full((gdst.shape[0], 16), 0.5, _F32)
    )
    cnt = jnp.stack([c1, c1])

    x = _fc1_call(X_list[0].astype(_F32), params["fc1"]["w"], params["fc1"]["b"])
    for _ in range(DEPTH):
        phi = _phi_call(x)
        xs = _sc_gather(phi, gsrc)
        msg = _edge_call(lmap, ea, xs, w1s, b1s, w2s, b2s, w3s, b3s)
        a1 = jnp.zeros((A_TOT, W), _F32).at[gdst].add(msg) * 0.5
        agg = jnp.stack([a1, a1])
        x = _combine_call(agg, cnt, phi, roots, cbias)

    return _fc23_call(
        x, params["fc2"]["w"], params["fc2"]["b"], params["fc3"]["w"],
        params["fc3"]["b"]
    )
